# Initial kernel scaffold; baseline (speedup 1.0000x reference)
import jax
import jax.numpy as jnp
from jax.experimental import pallas as pl


def kernel(indices, weight):
    raise NotImplementedError("write your pallas kernel here")


# speedup vs baseline: 22.9071x; 22.9071x over previous
import functools

import jax
import jax.numpy as jnp
from jax import lax
from jax.experimental import pallas as pl
from jax.experimental.pallas import tpu as pltpu
from jax.experimental.pallas import tpu_sc as plsc

_SEEDS = (179424941, 179425457, 179425907, 179426369)
_NUM_HASH = 4
_C = 200000
_D = 64
_B = 16384
_L = 20
_T = _B * _L
_NC = 2
_NS = 16
_NW = _NC * _NS
_CHUNK = 128
_TOK_PER_W = _T // _NW
_STEPS = _TOK_PER_W // _CHUNK
_LANES = 16


def _murmur_mod(idx_v):
    k0 = lax.bitcast_convert_type(idx_v, jnp.uint32)
    c1 = jnp.uint32(0xCC9E2D51)
    c2 = jnp.uint32(0x1B873593)
    k = k0 * c1
    k = (k << jnp.uint32(15)) | (k >> jnp.uint32(17))
    k = k * c2
    outs = []
    for seed in _SEEDS:
        h = jnp.full((_LANES,), seed, jnp.uint32) ^ k
        h = (h << jnp.uint32(13)) | (h >> jnp.uint32(19))
        h = h * jnp.uint32(5) + jnp.uint32(0xE6546B64)
        h = h ^ jnp.uint32(4)
        h = h ^ (h >> jnp.uint32(16))
        h = h * jnp.uint32(0x85EBCA6B)
        h = h ^ (h >> jnp.uint32(13))
        h = h * jnp.uint32(0xC2B2AE35)
        h = h ^ (h >> jnp.uint32(16))
        r = lax.bitcast_convert_type(h, jnp.int32)
        m = lax.rem(r, jnp.int32(_C))
        m = jnp.where(m < 0, m + jnp.int32(_C), m)
        m = jnp.where(idx_v == 0, jnp.int32(0), m)
        outs.append(m)
    return outs


@functools.partial(
    pl.kernel,
    out_type=jax.ShapeDtypeStruct((_T, _D), jnp.float32),
    mesh=plsc.VectorSubcoreMesh(core_axis_name="c", subcore_axis_name="s"),
    compiler_params=pltpu.CompilerParams(use_tc_tiling_on_sc=False),
    scratch_types=[
        pltpu.VMEM((_CHUNK,), jnp.int32),
        pltpu.VMEM((_NUM_HASH, _CHUNK), jnp.int32),
        pltpu.VMEM((_NUM_HASH, _CHUNK, _D), jnp.float32),
        pltpu.VMEM((_CHUNK, _D), jnp.float32),
        pltpu.SemaphoreType.DMA,
    ],
)
def _bloom(idx_hbm, w_hbm, out_hbm, idx_v, hidx, rows, out_v, sem):
    wid = lax.axis_index("s") * _NC + lax.axis_index("c")
    base = wid * _TOK_PER_W

    def step(s, carry):
        off = base + s * _CHUNK
        pltpu.sync_copy(idx_hbm.at[pl.ds(off, _CHUNK)], idx_v)
        for i in range(_CHUNK // _LANES):
            sl = pl.ds(i * _LANES, _LANES)
            hashes = _murmur_mod(idx_v[sl])
            for j in range(_NUM_HASH):
                hidx[j, sl] = hashes[j]
        cps = [
            pltpu.async_copy(w_hbm.at[hidx.at[j]], rows.at[j], sem)
            for j in range(_NUM_HASH)
        ]
        for cp in cps:
            cp.wait()

        def sum_body(t, c):
            for col in range(_D // _LANES):
                sl = pl.ds(col * _LANES, _LANES)
                out_v[t, sl] = (
                    rows[0, t, sl] + rows[1, t, sl]
                    + rows[2, t, sl] + rows[3, t, sl]
                )
            return c

        lax.fori_loop(0, _CHUNK, sum_body, None)
        pltpu.sync_copy(out_v, out_hbm.at[pl.ds(off, _CHUNK)])
        return carry

    lax.fori_loop(0, _STEPS, step, None)


def kernel(indices, weight):
    flat = indices.reshape(-1)
    out = _bloom(flat, weight)
    return out.reshape(indices.shape[0], indices.shape[1], _D)
